# baseline (speedup 1.0000x reference)
import functools

import jax
import jax.numpy as jnp
from jax import lax
from jax.experimental import pallas as pl
from jax.experimental.pallas import tpu as pltpu
from jax.experimental.pallas import tpu_sc as plsc

N = 10000
E = 320000
H = 128
LAT = 64
NODE_C = 64
EDGE_C = 8
STRUCT_D = 16
G = 64

NC = 2
NS = 16
NW = NC * NS
CH = 128
NBUF = 2
NG = 40
EWP = CH * NBUF * NG
EP = EWP * NW
AR = 10112
RPT = AR // NS

BN = 1000
NB = N // BN

_MESH = plsc.VectorSubcoreMesh(core_axis_name="c", subcore_axis_name="s",
                               num_cores=NC, num_subcores=NS)
_PREC = jax.lax.Precision.DEFAULT


def _dot(a, b):
    return jax.lax.dot_general(a, b, (((1,), (0,)), ((), ())), precision=_PREC)


def _dot_exact(a, b, dims=(((1,), (0,)), ((), ()))):
    return jax.lax.dot_general(a, b, dims, precision=jax.lax.Precision.HIGHEST)


@functools.cache
def _make_gather_scatter(table_rows):
    @functools.partial(
        pl.kernel,
        out_type=jax.ShapeDtypeStruct((NC, AR, H), jnp.float32),
        mesh=_MESH,
        scratch_types=(
            [pltpu.VMEM((NG // 2, NBUF, CH), jnp.int32),
             pltpu.VMEM((NG // 2, NBUF, CH), jnp.int32)]
            + [pltpu.VMEM((CH, H), jnp.float32) for _ in range(NBUF)]
            + [pltpu.VMEM_SHARED((AR, H), jnp.float32)]
            + [pltpu.SemaphoreType.DMA for _ in range(2 * NBUF)]
        ),
    )
    def gs(table, gidx, sidx, out, giv, siv, *rest):
        rows = rest[:NBUF]
        acc = rest[NBUF]
        gsems = rest[NBUF + 1:2 * NBUF + 1]
        ssems = rest[2 * NBUF + 1:3 * NBUF + 1]
        c = lax.axis_index("c")
        s = lax.axis_index("s")
        wid = c * NS + s

        z = jnp.zeros((16,), jnp.float32)

        def zero_row(r, carry):
            for j in range(H // 16):
                rows[0][r, pl.ds(j * 16, 16)] = z
            return carry

        lax.fori_loop(0, CH, zero_row, 0)
        base = s * RPT
        full = RPT // CH
        for k in range(full):
            pltpu.sync_copy(rows[0], acc.at[pl.ds(base + k * CH, CH)])
        rem = RPT - full * CH
        if rem:
            pltpu.sync_copy(rows[0].at[pl.ds(0, rem)],
                            acc.at[pl.ds(base + full * CH, rem)])

        plsc.subcore_barrier()

        gbase = wid * NG
        NG2 = NG // 2
        for half in range(2):
            pltpu.sync_copy(gidx.at[pl.ds(gbase + half * NG2, NG2)], giv)
            pltpu.sync_copy(sidx.at[pl.ds(gbase + half * NG2, NG2)], siv)

            def group(g, carry):
                gds = []
                for b in range(NBUF):
                    gds.append(pltpu.async_copy(table.at[giv.at[g, b]],
                                                rows[b], gsems[b]))
                sds = []
                for b in range(NBUF):
                    gds[b].wait()
                    sds.append(pltpu.async_copy(rows[b],
                                                acc.at[siv.at[g, b]],
                                                ssems[b], add=True))
                for b in range(NBUF):
                    sds[b].wait()
                return carry

            lax.fori_loop(0, NG2, group, 0)
        plsc.subcore_barrier()
        pltpu.sync_copy(acc.at[pl.ds(base, RPT)], out.at[c, pl.ds(base, RPT)])

    return gs


@functools.partial(
    pl.kernel,
    out_type=jax.ShapeDtypeStruct((EP, 16), jnp.float32),
    mesh=_MESH,
    scratch_types=(
        [pltpu.VMEM((NG, NBUF, CH), jnp.int32),
         pltpu.VMEM((NG, NBUF, CH), jnp.int32)]
        + [pltpu.VMEM((CH, H), jnp.float32) for _ in range(2 * NBUF)]
        + [pltpu.VMEM((CH, 16), jnp.float32)]
        + [pltpu.VMEM((16,), jnp.float32)]
        + [pltpu.SemaphoreType.DMA for _ in range(2 * NBUF)]
    ),
)
def _edge_logits_sc(hw, b16, sidxg, didxg, out, svv, dvv, *rest):
    ra = rest[:NBUF]
    rb = rest[NBUF:2 * NBUF]
    ov = rest[2 * NBUF]
    bv = rest[2 * NBUF + 1]
    sa = rest[2 * NBUF + 2:2 * NBUF + 2 + NBUF]
    sb = rest[2 * NBUF + 2 + NBUF:]
    c = lax.axis_index("c")
    s = lax.axis_index("s")
    wid = c * NS + s
    pltpu.sync_copy(sidxg.at[pl.ds(wid * NG, NG)], svv)
    pltpu.sync_copy(didxg.at[pl.ds(wid * NG, NG)], dvv)
    pltpu.sync_copy(b16, bv)
    bias = bv[...]

    def group(g, carry):
        gds = []
        for b in range(NBUF):
            gds.append(pltpu.async_copy(hw.at[svv.at[g, b]], ra[b], sa[b]))
            gds.append(pltpu.async_copy(hw.at[dvv.at[g, b]], rb[b], sb[b]))
        for b in range(NBUF):
            gds[2 * b].wait()
            gds[2 * b + 1].wait()

            def addrow(r, cc, _b=b):
                ov[r, :] = (ra[_b][r, pl.ds(0, 16)]
                            + rb[_b][r, pl.ds(0, 16)] + bias)
                return cc

            lax.fori_loop(0, CH, addrow, 0)
            pltpu.sync_copy(
                ov, out.at[pl.ds(wid * EWP + (g * NBUF + b) * CH, CH)])
        return carry

    lax.fori_loop(0, NG, group, 0)


def _comb_body(b_ref, s_ref, o_ref):
    o_ref[...] = b_ref[...] * N + s_ref[...]


def _feat_body(oa_ref, ob_ref, deg_ref, Wn_ref, Ws_ref, bs_ref, x_ref, cnt_ref):
    i = pl.program_id(0)
    x_ref[...] = (_dot_exact(oa_ref[...], Wn_ref[...])
                  + _dot(deg_ref[...], Ws_ref[...]) + bs_ref[...])

    @pl.when(i == 0)
    def _():
        cnt_ref[...] = jnp.zeros_like(cnt_ref)

    cnt_ref[...] += jnp.sum(ob_ref[...], axis=0, keepdims=True)

    @pl.when(i == NB - 1)
    def _():
        cnt_ref[...] = jnp.maximum(cnt_ref[...], 1.0)


def _ypool_body(x_ref, emb_ref, ob_ref, y_ref, pooled_ref):
    i = pl.program_id(0)
    x = x_ref[...]
    for b in range(EDGE_C):
        y_ref[b] = jnp.maximum(x + emb_ref[b:b + 1, :], 0.0)

    @pl.when(i == 0)
    def _():
        pooled_ref[...] = jnp.zeros_like(pooled_ref)

    pooled_ref[...] += _dot_exact(ob_ref[...], x, (((0,), (0,)), ((), ())))


def _encup_body(x_ref, p_ref, pooled_ref, cnt_ref, ob_ref, Wg_ref, W_ref,
                b_ref, o_ref):
    oh = ob_ref[...]
    pg = _dot_exact(oh, pooled_ref[...])
    cnt_node = jnp.sum(oh * cnt_ref[...], axis=1, keepdims=True)
    glob = _dot(pg / cnt_node, Wg_ref[...])
    sfull = x_ref[...] + p_ref[0] + p_ref[1] + glob
    o_ref[...] = jnp.maximum(_dot(sfull, W_ref[...]) + b_ref[...], 0.0)


def _heads_body(x_ref, eps_ref, deg_ref, Wmu_ref, bmu_ref, Wls_ref, bls_ref,
                Wdi_ref, Wsd_ref, mu_ref, ls_ref, h0_ref):
    x = x_ref[...]
    mu = _dot(x, Wmu_ref[...]) + bmu_ref[...]
    ls = _dot(x, Wls_ref[...]) + bls_ref[...]
    mu_ref[...] = mu
    ls_ref[...] = ls
    smp = jnp.exp(ls) * eps_ref[...] + mu
    h0_ref[...] = jnp.maximum(_dot(smp, Wdi_ref[...])
                              + _dot(deg_ref[...], Wsd_ref[...]), 0.0)


def _decup_body(h_ref, p_ref, W_ref, b_ref, o_ref):
    sfull = h_ref[...] + p_ref[0] + p_ref[1]
    o_ref[...] = jnp.maximum(_dot(sfull, W_ref[...]) + b_ref[...], 0.0)


def _final_body(h_ref, Wno_ref, bno_ref, We_ref, nl_ref, hw_ref):
    h = h_ref[...]
    nl_ref[...] = _dot(h, Wno_ref[...]) + bno_ref[...]
    hw_ref[...] = _dot(h, We_ref[...])


def _bs(shape, index_map):
    return pl.BlockSpec(shape, index_map)


_I0 = lambda i: (i, 0)
_C0 = lambda i: (0, 0)


def kernel(atom_type, bond_type, degree_feat, edge_index, batch_vec, W_node_emb, W_edge_emb, W_struct, b_struct, enc_W, enc_b, enc_Wg, W_mu, b_mu, W_ls, b_ls, W_dec_in, W_struct_dec, dec_W, dec_b, W_node_out, b_node_out, W_edge_out, b_edge_out, eps):
    f32 = jnp.float32
    atom = atom_type.astype(jnp.int32)
    bond = bond_type.astype(jnp.int32)
    src = edge_index[0].astype(jnp.int32)
    dst = edge_index[1].astype(jnp.int32)
    batch = batch_vec.astype(jnp.int32)
    L_ENC = enc_W.shape[0]
    L_DEC = dec_W.shape[0]

    oh_atom = jax.nn.one_hot(atom, NODE_C, dtype=f32)
    oh_batch = jax.nn.one_hot(batch, G, dtype=f32)

    padlen = EP - E
    src_p = jnp.concatenate([src, jnp.zeros((padlen,), jnp.int32)])
    dst_p = jnp.concatenate([dst, jnp.full((padlen,), N, jnp.int32)])
    bond_p = jnp.concatenate([bond, jnp.zeros((padlen,), jnp.int32)])

    comb = pl.pallas_call(
        _comb_body,
        out_shape=jax.ShapeDtypeStruct((EP // CH, CH), jnp.int32),
    )(bond_p.reshape(EP // CH, CH), src_p.reshape(EP // CH, CH))
    gidx_enc = comb.reshape(NW * NG, NBUF, CH)
    gidx_dec = src_p.reshape(NW * NG, NBUF, CH)
    sidx = dst_p.reshape(NW * NG, NBUF, CH)

    x, cnt = pl.pallas_call(
        _feat_body,
        grid=(NB,),
        in_specs=[
            _bs((BN, NODE_C), _I0), _bs((BN, G), _I0), _bs((BN, STRUCT_D), _I0),
            _bs((NODE_C, H), _C0), _bs((STRUCT_D, H), _C0), _bs((1, H), _C0),
        ],
        out_specs=[_bs((BN, H), _I0), _bs((1, G), _C0)],
        out_shape=[jax.ShapeDtypeStruct((N, H), f32),
                   jax.ShapeDtypeStruct((1, G), f32)],
    )(oh_atom, oh_batch, degree_feat, W_node_emb, W_struct,
      b_struct.reshape(1, H))

    gs_enc = _make_gather_scatter(EDGE_C * N)
    for i in range(L_ENC):
        y, pooled = pl.pallas_call(
            _ypool_body,
            grid=(NB,),
            in_specs=[
                _bs((BN, H), _I0), _bs((EDGE_C, H), _C0), _bs((BN, G), _I0),
            ],
            out_specs=[pl.BlockSpec((EDGE_C, BN, H), lambda i: (0, i, 0)),
                       _bs((G, H), _C0)],
            out_shape=[jax.ShapeDtypeStruct((EDGE_C, N, H), f32),
                       jax.ShapeDtypeStruct((G, H), f32)],
        )(x, W_edge_emb, oh_batch)
        part = gs_enc(y.reshape(EDGE_C * N, H), gidx_enc, sidx)
        x = pl.pallas_call(
            _encup_body,
            grid=(NB,),
            in_specs=[
                _bs((BN, H), _I0),
                pl.BlockSpec((NC, BN, H), lambda i: (0, i, 0)),
                _bs((G, H), _C0), _bs((1, G), _C0), _bs((BN, G), _I0),
                _bs((H, H), _C0), _bs((H, H), _C0), _bs((1, H), _C0),
            ],
            out_specs=_bs((BN, H), _I0),
            out_shape=jax.ShapeDtypeStruct((N, H), f32),
        )(x, part, pooled, cnt, oh_batch, enc_Wg[i], enc_W[i],
          enc_b[i].reshape(1, H))

    mu, logsigma, h = pl.pallas_call(
        _heads_body,
        grid=(NB,),
        in_specs=[
            _bs((BN, H), _I0), _bs((BN, LAT), _I0), _bs((BN, STRUCT_D), _I0),
            _bs((H, LAT), _C0), _bs((1, LAT), _C0),
            _bs((H, LAT), _C0), _bs((1, LAT), _C0),
            _bs((LAT, H), _C0), _bs((STRUCT_D, H), _C0),
        ],
        out_specs=[_bs((BN, LAT), _I0), _bs((BN, LAT), _I0), _bs((BN, H), _I0)],
        out_shape=[jax.ShapeDtypeStruct((N, LAT), f32),
                   jax.ShapeDtypeStruct((N, LAT), f32),
                   jax.ShapeDtypeStruct((N, H), f32)],
    )(x, eps, degree_feat, W_mu, b_mu.reshape(1, LAT), W_ls,
      b_ls.reshape(1, LAT), W_dec_in, W_struct_dec)

    gs_dec = _make_gather_scatter(N)
    for i in range(L_DEC):
        part = gs_dec(h, gidx_dec, sidx)
        h = pl.pallas_call(
            _decup_body,
            grid=(NB,),
            in_specs=[
                _bs((BN, H), _I0),
                pl.BlockSpec((NC, BN, H), lambda i: (0, i, 0)),
                _bs((H, H), _C0), _bs((1, H), _C0),
            ],
            out_specs=_bs((BN, H), _I0),
            out_shape=jax.ShapeDtypeStruct((N, H), f32),
        )(h, part, dec_W[i], dec_b[i].reshape(1, H))

    node_logits, hw = pl.pallas_call(
        _final_body,
        grid=(NB,),
        in_specs=[
            _bs((BN, H), _I0), _bs((H, NODE_C), _C0), _bs((1, NODE_C), _C0),
            _bs((H, H), _C0),
        ],
        out_specs=[_bs((BN, NODE_C), _I0), _bs((BN, H), _I0)],
        out_shape=[jax.ShapeDtypeStruct((N, NODE_C), f32),
                   jax.ShapeDtypeStruct((N, H), f32)],
    )(h, W_node_out, b_node_out.reshape(1, NODE_C),
      jnp.pad(W_edge_out, ((0, 0), (0, H - EDGE_C))))

    hw_pad = jnp.pad(hw, ((0, AR - N), (0, 0)))
    b16 = jnp.pad(b_edge_out, (0, 16 - EDGE_C))
    el16 = _edge_logits_sc(hw_pad, b16, gidx_dec, sidx)
    edge_logits = el16[:E, :EDGE_C]

    return (mu, logsigma, node_logits, edge_logits)


# speedup vs baseline: 1.4307x; 1.4307x over previous
import functools

import jax
import jax.numpy as jnp
from jax import lax
from jax.experimental import pallas as pl
from jax.experimental.pallas import tpu as pltpu
from jax.experimental.pallas import tpu_sc as plsc

N = 10000
E = 320000
H = 128
LAT = 64
NODE_C = 64
EDGE_C = 8
STRUCT_D = 16
G = 64

NC = 2
NS = 16
NW = NC * NS
CH = 128
NCH = 79
EWP = CH * NCH
EP = EWP * NW
AR = 10112
RPT = AR // NS

BN = 1000
NB = N // BN

_MESH = plsc.VectorSubcoreMesh(core_axis_name="c", subcore_axis_name="s",
                               num_cores=NC, num_subcores=NS)
_PREC = jax.lax.Precision.DEFAULT


def _dot(a, b):
    return jax.lax.dot_general(a, b, (((1,), (0,)), ((), ())), precision=_PREC)


def _dot_exact(a, b, dims=(((1,), (0,)), ((), ()))):
    return jax.lax.dot_general(a, b, dims, precision=jax.lax.Precision.HIGHEST)


@functools.cache
def _make_gather_scatter(table_rows):
    @functools.partial(
        pl.kernel,
        out_type=jax.ShapeDtypeStruct((NC, AR, H), jnp.float32),
        mesh=_MESH,
        scratch_types=[
            pltpu.VMEM((NCH, CH), jnp.int32),
            pltpu.VMEM((NCH, CH), jnp.int32),
            pltpu.VMEM((CH, H), jnp.float32),
            pltpu.VMEM_SHARED((AR, H), jnp.float32),
            pltpu.SemaphoreType.DMA,
        ],
    )
    def gs(table, gidx, sidx, out, gv, sv, rows, acc, sem):
        c = lax.axis_index("c")
        s = lax.axis_index("s")
        wid = c * NS + s

        z = jnp.zeros((16,), jnp.float32)

        def zero_row(r, carry):
            for j in range(H // 16):
                rows[r, pl.ds(j * 16, 16)] = z
            return carry

        lax.fori_loop(0, CH, zero_row, 0)
        base = s * RPT
        full = RPT // CH
        for k in range(full):
            pltpu.sync_copy(rows, acc.at[pl.ds(base + k * CH, CH)])
        rem = RPT - full * CH
        if rem:
            pltpu.sync_copy(rows.at[pl.ds(0, rem)],
                            acc.at[pl.ds(base + full * CH, rem)])

        pltpu.sync_copy(gidx.at[wid], gv)
        pltpu.sync_copy(sidx.at[wid], sv)
        plsc.subcore_barrier()

        def step(j, carry):
            pltpu.async_copy(table.at[gv.at[j, :]], rows, sem).wait()
            pltpu.sync_copy(rows, acc.at[sv.at[j, :]], add=True)
            return carry

        lax.fori_loop(0, NCH, step, 0)
        plsc.subcore_barrier()
        pltpu.sync_copy(acc.at[pl.ds(base, RPT)], out.at[c, pl.ds(base, RPT)])

    return gs


@functools.partial(
    pl.kernel,
    out_type=jax.ShapeDtypeStruct((EP, 16), jnp.float32),
    mesh=_MESH,
    scratch_types=[
        pltpu.VMEM((NCH, CH), jnp.int32),
        pltpu.VMEM((NCH, CH), jnp.int32),
        pltpu.VMEM((CH, H), jnp.float32),
        pltpu.VMEM((CH, H), jnp.float32),
        pltpu.VMEM((CH, 16), jnp.float32),
        pltpu.VMEM((16,), jnp.float32),
        pltpu.SemaphoreType.DMA,
        pltpu.SemaphoreType.DMA,
    ],
)
def _edge_logits_sc(hw, b16, sidxg, didxg, out, svv, dvv, ra, rb, ov, bv,
                    sema, semb):
    c = lax.axis_index("c")
    s = lax.axis_index("s")
    wid = c * NS + s
    pltpu.sync_copy(sidxg.at[wid], svv)
    pltpu.sync_copy(didxg.at[wid], dvv)
    pltpu.sync_copy(b16, bv)
    bias = bv[...]

    def step(j, carry):
        da = pltpu.async_copy(hw.at[svv.at[j, :]], ra, sema)
        db = pltpu.async_copy(hw.at[dvv.at[j, :]], rb, semb)
        da.wait()
        db.wait()

        def addrow(r, cc):
            ov[r, :] = ra[r, pl.ds(0, 16)] + rb[r, pl.ds(0, 16)] + bias
            return cc

        lax.fori_loop(0, CH, addrow, 0)
        pltpu.sync_copy(ov, out.at[pl.ds(wid * EWP + j * CH, CH)])
        return carry

    lax.fori_loop(0, NCH, step, 0)


def _comb_body(b_ref, s_ref, o_ref):
    o_ref[...] = b_ref[...] * N + s_ref[...]


def _feat_body(oa_ref, ob_ref, deg_ref, Wn_ref, Ws_ref, bs_ref, x_ref, cnt_ref):
    i = pl.program_id(0)
    x_ref[...] = (_dot_exact(oa_ref[...], Wn_ref[...])
                  + _dot(deg_ref[...], Ws_ref[...]) + bs_ref[...])

    @pl.when(i == 0)
    def _():
        cnt_ref[...] = jnp.zeros_like(cnt_ref)

    cnt_ref[...] += jnp.sum(ob_ref[...], axis=0, keepdims=True)

    @pl.when(i == NB - 1)
    def _():
        cnt_ref[...] = jnp.maximum(cnt_ref[...], 1.0)


def _ypool_body(x_ref, emb_ref, ob_ref, y_ref, pooled_ref):
    i = pl.program_id(0)
    x = x_ref[...]
    for b in range(EDGE_C):
        y_ref[b] = jnp.maximum(x + emb_ref[b:b + 1, :], 0.0)

    @pl.when(i == 0)
    def _():
        pooled_ref[...] = jnp.zeros_like(pooled_ref)

    pooled_ref[...] += _dot_exact(ob_ref[...], x, (((0,), (0,)), ((), ())))


def _encup_body(x_ref, p_ref, pooled_ref, cnt_ref, ob_ref, Wg_ref, W_ref,
                b_ref, o_ref):
    oh = ob_ref[...]
    pg = _dot_exact(oh, pooled_ref[...])
    cnt_node = jnp.sum(oh * cnt_ref[...], axis=1, keepdims=True)
    glob = _dot(pg / cnt_node, Wg_ref[...])
    sfull = x_ref[...] + p_ref[0] + p_ref[1] + glob
    o_ref[...] = jnp.maximum(_dot(sfull, W_ref[...]) + b_ref[...], 0.0)


def _heads_body(x_ref, eps_ref, deg_ref, Wmu_ref, bmu_ref, Wls_ref, bls_ref,
                Wdi_ref, Wsd_ref, mu_ref, ls_ref, h0_ref):
    x = x_ref[...]
    mu = _dot(x, Wmu_ref[...]) + bmu_ref[...]
    ls = _dot(x, Wls_ref[...]) + bls_ref[...]
    mu_ref[...] = mu
    ls_ref[...] = ls
    smp = jnp.exp(ls) * eps_ref[...] + mu
    h0_ref[...] = jnp.maximum(_dot(smp, Wdi_ref[...])
                              + _dot(deg_ref[...], Wsd_ref[...]), 0.0)


def _decup_body(h_ref, p_ref, W_ref, b_ref, o_ref):
    sfull = h_ref[...] + p_ref[0] + p_ref[1]
    o_ref[...] = jnp.maximum(_dot(sfull, W_ref[...]) + b_ref[...], 0.0)


def _final_body(h_ref, Wno_ref, bno_ref, We_ref, nl_ref, hw_ref):
    h = h_ref[...]
    nl_ref[...] = _dot(h, Wno_ref[...]) + bno_ref[...]
    hw_ref[...] = _dot(h, We_ref[...])


def _bs(shape, index_map):
    return pl.BlockSpec(shape, index_map)


_I0 = lambda i: (i, 0)
_C0 = lambda i: (0, 0)


def kernel(atom_type, bond_type, degree_feat, edge_index, batch_vec, W_node_emb, W_edge_emb, W_struct, b_struct, enc_W, enc_b, enc_Wg, W_mu, b_mu, W_ls, b_ls, W_dec_in, W_struct_dec, dec_W, dec_b, W_node_out, b_node_out, W_edge_out, b_edge_out, eps):
    f32 = jnp.float32
    atom = atom_type.astype(jnp.int32)
    bond = bond_type.astype(jnp.int32)
    src = edge_index[0].astype(jnp.int32)
    dst = edge_index[1].astype(jnp.int32)
    batch = batch_vec.astype(jnp.int32)
    L_ENC = enc_W.shape[0]
    L_DEC = dec_W.shape[0]

    oh_atom = jax.nn.one_hot(atom, NODE_C, dtype=f32)
    oh_batch = jax.nn.one_hot(batch, G, dtype=f32)

    padlen = EP - E
    src_p = jnp.concatenate([src, jnp.zeros((padlen,), jnp.int32)])
    dst_p = jnp.concatenate([dst, jnp.full((padlen,), N, jnp.int32)])
    bond_p = jnp.concatenate([bond, jnp.zeros((padlen,), jnp.int32)])

    comb = pl.pallas_call(
        _comb_body,
        out_shape=jax.ShapeDtypeStruct((EP // CH, CH), jnp.int32),
    )(bond_p.reshape(EP // CH, CH), src_p.reshape(EP // CH, CH))
    gidx_enc = comb.reshape(NW, NCH, CH)
    gidx_dec = src_p.reshape(NW, NCH, CH)
    sidx = dst_p.reshape(NW, NCH, CH)

    x, cnt = pl.pallas_call(
        _feat_body,
        grid=(NB,),
        in_specs=[
            _bs((BN, NODE_C), _I0), _bs((BN, G), _I0), _bs((BN, STRUCT_D), _I0),
            _bs((NODE_C, H), _C0), _bs((STRUCT_D, H), _C0), _bs((1, H), _C0),
        ],
        out_specs=[_bs((BN, H), _I0), _bs((1, G), _C0)],
        out_shape=[jax.ShapeDtypeStruct((N, H), f32),
                   jax.ShapeDtypeStruct((1, G), f32)],
    )(oh_atom, oh_batch, degree_feat, W_node_emb, W_struct,
      b_struct.reshape(1, H))

    gs_enc = _make_gather_scatter(EDGE_C * N)
    for i in range(L_ENC):
        y, pooled = pl.pallas_call(
            _ypool_body,
            grid=(NB,),
            in_specs=[
                _bs((BN, H), _I0), _bs((EDGE_C, H), _C0), _bs((BN, G), _I0),
            ],
            out_specs=[pl.BlockSpec((EDGE_C, BN, H), lambda i: (0, i, 0)),
                       _bs((G, H), _C0)],
            out_shape=[jax.ShapeDtypeStruct((EDGE_C, N, H), f32),
                       jax.ShapeDtypeStruct((G, H), f32)],
        )(x, W_edge_emb, oh_batch)
        part = gs_enc(y.reshape(EDGE_C * N, H), gidx_enc, sidx)
        x = pl.pallas_call(
            _encup_body,
            grid=(NB,),
            in_specs=[
                _bs((BN, H), _I0),
                pl.BlockSpec((NC, BN, H), lambda i: (0, i, 0)),
                _bs((G, H), _C0), _bs((1, G), _C0), _bs((BN, G), _I0),
                _bs((H, H), _C0), _bs((H, H), _C0), _bs((1, H), _C0),
            ],
            out_specs=_bs((BN, H), _I0),
            out_shape=jax.ShapeDtypeStruct((N, H), f32),
        )(x, part, pooled, cnt, oh_batch, enc_Wg[i], enc_W[i],
          enc_b[i].reshape(1, H))

    mu, logsigma, h = pl.pallas_call(
        _heads_body,
        grid=(NB,),
        in_specs=[
            _bs((BN, H), _I0), _bs((BN, LAT), _I0), _bs((BN, STRUCT_D), _I0),
            _bs((H, LAT), _C0), _bs((1, LAT), _C0),
            _bs((H, LAT), _C0), _bs((1, LAT), _C0),
            _bs((LAT, H), _C0), _bs((STRUCT_D, H), _C0),
        ],
        out_specs=[_bs((BN, LAT), _I0), _bs((BN, LAT), _I0), _bs((BN, H), _I0)],
        out_shape=[jax.ShapeDtypeStruct((N, LAT), f32),
                   jax.ShapeDtypeStruct((N, LAT), f32),
                   jax.ShapeDtypeStruct((N, H), f32)],
    )(x, eps, degree_feat, W_mu, b_mu.reshape(1, LAT), W_ls,
      b_ls.reshape(1, LAT), W_dec_in, W_struct_dec)

    gs_dec = _make_gather_scatter(N)
    for i in range(L_DEC):
        part = gs_dec(h, gidx_dec, sidx)
        h = pl.pallas_call(
            _decup_body,
            grid=(NB,),
            in_specs=[
                _bs((BN, H), _I0),
                pl.BlockSpec((NC, BN, H), lambda i: (0, i, 0)),
                _bs((H, H), _C0), _bs((1, H), _C0),
            ],
            out_specs=_bs((BN, H), _I0),
            out_shape=jax.ShapeDtypeStruct((N, H), f32),
        )(h, part, dec_W[i], dec_b[i].reshape(1, H))

    node_logits, hw = pl.pallas_call(
        _final_body,
        grid=(NB,),
        in_specs=[
            _bs((BN, H), _I0), _bs((H, NODE_C), _C0), _bs((1, NODE_C), _C0),
            _bs((H, H), _C0),
        ],
        out_specs=[_bs((BN, NODE_C), _I0), _bs((BN, H), _I0)],
        out_shape=[jax.ShapeDtypeStruct((N, NODE_C), f32),
                   jax.ShapeDtypeStruct((N, H), f32)],
    )(h, W_node_out, b_node_out.reshape(1, NODE_C),
      jnp.pad(W_edge_out, ((0, 0), (0, H - EDGE_C))))

    hw_pad = jnp.pad(hw, ((0, AR - N), (0, 0)))
    b16 = jnp.pad(b_edge_out, (0, 16 - EDGE_C))
    el16 = _edge_logits_sc(hw_pad, b16, gidx_dec, sidx)
    edge_logits = el16[:E, :EDGE_C]

    return (mu, logsigma, node_logits, edge_logits)
